# baseline (speedup 1.0000x reference)
import jax
import jax.numpy as jnp
from jax import lax
from jax.experimental import pallas as pl
from jax.experimental.pallas import tpu as pltpu

_SB = 4


def _scale_body(packed_ref, mags_ref, x_ref, o_ref):
    i = pl.program_id(0)
    w = packed_ref[i]
    for j in range(_SB):
        b = i * _SB + j
        byte = (w >> (8 * j)) & 0xFF
        scale = jnp.where(byte != 0, mags_ref[b], jnp.float32(1.0))
        o_ref[j] = x_ref[j] * scale


def kernel(input, probs, magnitudes):
    B, C, H, W = input.shape
    packed = lax.bitcast_convert_type(
        probs.view(jnp.int8).reshape(B // 4, 4), jnp.int32
    )
    out = pl.pallas_call(
        _scale_body,
        grid_spec=pltpu.PrefetchScalarGridSpec(
            num_scalar_prefetch=2,
            grid=(B // _SB,),
            in_specs=[pl.BlockSpec((_SB, C, H, W), lambda i, p, m: (i, 0, 0, 0))],
            out_specs=pl.BlockSpec((_SB, C, H, W), lambda i, p, m: (i, 0, 0, 0)),
        ),
        out_shape=jax.ShapeDtypeStruct((B, C, H, W), jnp.float32),
    )(packed, magnitudes, input)
    return out


# speedup vs baseline: 1.0015x; 1.0015x over previous
import jax
import jax.numpy as jnp
from jax import lax
from jax.experimental import pallas as pl
from jax.experimental.pallas import tpu as pltpu

_SB = 4


def _scale_body(probs_ref, mags_ref, x_ref, o_ref):
    i = pl.program_id(0)
    for j in range(_SB):
        b = i * _SB + j
        p = probs_ref[b].astype(jnp.int32)
        scale = jnp.where(p != 0, mags_ref[b], jnp.float32(1.0))
        o_ref[j] = x_ref[j] * scale


def kernel(input, probs, magnitudes):
    B, C, H, W = input.shape
    packed = probs.view(jnp.int8)
    out = pl.pallas_call(
        _scale_body,
        grid_spec=pltpu.PrefetchScalarGridSpec(
            num_scalar_prefetch=2,
            grid=(B // _SB,),
            in_specs=[pl.BlockSpec((_SB, C, H, W), lambda i, p, m: (i, 0, 0, 0))],
            out_specs=pl.BlockSpec((_SB, C, H, W), lambda i, p, m: (i, 0, 0, 0)),
        ),
        out_shape=jax.ShapeDtypeStruct((B, C, H, W), jnp.float32),
    )(packed, magnitudes, input)
    return out


# speedup vs baseline: 1.0077x; 1.0063x over previous
import jax
import jax.numpy as jnp
from jax.experimental import pallas as pl
from jax.experimental.pallas import tpu as pltpu

_SB = 4


def _scale_body(probs_ref, mags_ref, x_ref, o_ref):
    i = pl.program_id(0)
    sv = jnp.where(probs_ref[...], mags_ref[...], jnp.float32(1.0))
    lane = jax.lax.broadcasted_iota(jnp.int32, sv.shape, 0)
    for j in range(_SB):
        b = i * _SB + j
        scale = jnp.sum(jnp.where(lane == b, sv, jnp.float32(0.0)))
        o_ref[j] = x_ref[j] * scale


def kernel(input, probs, magnitudes):
    B, C, H, W = input.shape
    out = pl.pallas_call(
        _scale_body,
        grid=(B // _SB,),
        in_specs=[
            pl.BlockSpec((B,), lambda i: (0,)),
            pl.BlockSpec((B,), lambda i: (0,)),
            pl.BlockSpec((_SB, C, H, W), lambda i: (i, 0, 0, 0)),
        ],
        out_specs=pl.BlockSpec((_SB, C, H, W), lambda i: (i, 0, 0, 0)),
        out_shape=jax.ShapeDtypeStruct((B, C, H, W), jnp.float32),
    )(probs, magnitudes, input)
    return out
